# baseline (speedup 1.0000x reference)
import functools

import jax
import jax.numpy as jnp
from jax import lax
from jax.experimental import pallas as pl
from jax.experimental.pallas import tpu as pltpu
from jax.experimental.pallas import tpu_sc as plsc

_B, _S, _D, _DFF, _E, _TOPK = 1, 2048, 1024, 4096, 8, 2
_EPS = 1e-5
_BLK = 128
_NP = _S * _TOPK
_P = _NP + _E * _BLK
_NB = _P // _BLK
_NC, _NS = 2, 16
_NW = _NC * _NS


def _wid():
    return lax.axis_index("s") * _NC + lax.axis_index("c")


_PH = _P // 2
_GPW = _PH // _NW
_GCH = _GPW // 2


@functools.cache
def _make_sc_gather():
    @functools.partial(
        pl.kernel,
        mesh=plsc.VectorSubcoreMesh(core_axis_name="c", subcore_axis_name="s"),
        out_type=jax.ShapeDtypeStruct((_PH, _D), jnp.float32),
        scratch_types=[
            pltpu.VMEM((_GPW,), jnp.int32),
            pltpu.VMEM((_GCH, _D), jnp.float32),
            pltpu.VMEM((_GCH, _D), jnp.float32),
            pltpu.SemaphoreType.DMA,
            pltpu.SemaphoreType.DMA,
        ],
    )
    def _sc_gather(x_hbm, rt_hbm, xs_hbm, idx_v, b0, b1, s0, s1):
        w = _wid()
        base = w * _GPW
        pltpu.sync_copy(rt_hbm.at[pl.ds(base, _GPW)], idx_v)
        bufs, sems = (b0, b1), (s0, s1)
        cps = [None, None]
        for c in range(2):
            cps[c % 2] = pltpu.async_copy(
                x_hbm.at[idx_v.at[pl.ds(c * _GCH, _GCH)]],
                bufs[c % 2], sems[c % 2])
            if c >= 1:
                cps[(c - 1) % 2].wait()
                pltpu.sync_copy(bufs[(c - 1) % 2],
                                xs_hbm.at[pl.ds(base + (c - 1) * _GCH, _GCH)])
        cps[1].wait()
        pltpu.sync_copy(bufs[1], xs_hbm.at[pl.ds(base + _GCH, _GCH)])

    return _sc_gather


_CCH = 32


@functools.cache
def _make_sc_combine():
    @functools.partial(
        pl.kernel,
        mesh=plsc.VectorSubcoreMesh(core_axis_name="c", subcore_axis_name="s"),
        out_type=jax.ShapeDtypeStruct((_S, _D), jnp.float32),
        scratch_types=[
            pltpu.VMEM((_CCH,), jnp.int32),
            pltpu.VMEM((_CCH,), jnp.int32),
            pltpu.VMEM((_CCH, _D), jnp.float32),
            pltpu.VMEM((_CCH, _D), jnp.float32),
            pltpu.SemaphoreType.DMA,
            pltpu.SemaphoreType.DMA,
        ],
    )
    def _sc_combine(y_hbm, p0_hbm, p1_hbm, out_hbm, i0_v, i1_v, a_v, b_v,
                    s0, s1):
        w = _wid()
        for c in range(2):
            base = w * (2 * _CCH) + c * _CCH
            pltpu.sync_copy(p0_hbm.at[pl.ds(base, _CCH)], i0_v)
            pltpu.sync_copy(p1_hbm.at[pl.ds(base, _CCH)], i1_v)
            cp0 = pltpu.async_copy(y_hbm.at[i0_v], a_v, s0)
            cp1 = pltpu.async_copy(y_hbm.at[i1_v], b_v, s1)
            cp0.wait()
            cp1.wait()

            def _row(r, carry):
                for j in range(_D // 16):
                    sl = pl.ds(j * 16, 16)
                    a_v[r, sl] = a_v[r, sl] + b_v[r, sl]
                return carry

            lax.fori_loop(0, _CCH, _row, 0)
            pltpu.sync_copy(a_v, out_hbm.at[pl.ds(base, _CCH)])

    return _sc_combine


def _cast_body(b_ref, bo_ref):
    bo_ref[...] = b_ref[...].astype(jnp.bfloat16)


def _cast_w2(w2):
    w2f = w2.reshape(_E * _DFF, _D)
    n = 32
    r2 = w2f.shape[0] // n
    o2 = pl.pallas_call(
        _cast_body,
        grid=(n,),
        in_specs=[pl.BlockSpec((r2, _D), lambda i: (i, 0))],
        out_specs=pl.BlockSpec((r2, _D), lambda i: (i, 0)),
        out_shape=jax.ShapeDtypeStruct(w2f.shape, jnp.bfloat16),
        compiler_params=pltpu.CompilerParams(
            dimension_semantics=("arbitrary",)),
    )(w2f)
    return o2.reshape(_E, _DFF, _D)


_NBH = _NB // 2


def _ffn_body(off, meta_ref, xs_ref, wrow_ref, w1_ref, b1_ref, w2_ref,
              b2_ref, ls_ref, lb_ref, y_ref):
    i = pl.program_id(0) + off
    n_act = meta_ref[_NB]

    @pl.when(i < n_act)
    def _():
        x = xs_ref[...].astype(jnp.float32)
        mu = jnp.mean(x, axis=1, keepdims=True)
        xc = x - mu
        var = jnp.mean(xc * xc, axis=1, keepdims=True)
        xn = xc * lax.rsqrt(var + _EPS)
        h = xn * ls_ref[0] + lb_ref[0]
        a = jnp.dot(h, w1_ref[0],
                    preferred_element_type=jnp.float32) + b1_ref[0]
        g = jax.nn.gelu(a)
        y = jnp.dot(g.astype(jnp.bfloat16), w2_ref[0],
                    preferred_element_type=jnp.float32) + b2_ref[0]
        y_ref[...] = y * wrow_ref[:, :1]


def _ffn_half(off, meta, xs, wrow, w1, b1, w2, b2, ls, lb):
    grid_spec = pltpu.PrefetchScalarGridSpec(
        num_scalar_prefetch=1,
        grid=(_NBH,),
        in_specs=[
            pl.BlockSpec((_BLK, _D), lambda i, m: (i, 0)),
            pl.BlockSpec((_BLK, 128), lambda i, m: (i + off * _NBH, 0)),
            pl.BlockSpec((1, _D, _DFF), lambda i, m: (m[i + off * _NBH], 0, 0)),
            pl.BlockSpec((1, 1, _DFF), lambda i, m: (m[i + off * _NBH], 0, 0)),
            pl.BlockSpec((1, _DFF, _D), lambda i, m: (m[i + off * _NBH], 0, 0)),
            pl.BlockSpec((1, 1, _D), lambda i, m: (m[i + off * _NBH], 0, 0)),
            pl.BlockSpec((1, 1, _D), lambda i, m: (m[i + off * _NBH], 0, 0)),
            pl.BlockSpec((1, 1, _D), lambda i, m: (m[i + off * _NBH], 0, 0)),
        ],
        out_specs=pl.BlockSpec((_BLK, _D), lambda i, m: (i, 0)),
    )
    return pl.pallas_call(
        functools.partial(_ffn_body, off * _NBH),
        grid_spec=grid_spec,
        out_shape=jax.ShapeDtypeStruct((_PH, _D), jnp.float32),
        compiler_params=pltpu.CompilerParams(
            dimension_semantics=("arbitrary",)),
    )(meta, xs, wrow, w1, b1, w2, b2, ls, lb)


def kernel(hidden_states, gate_w, alpha, ln_scale, ln_bias, fc1_w, fc1_b,
           fc2_w, fc2_b):
    x = hidden_states.reshape(_S, _D)

    logits = x @ gate_w
    probs = jax.nn.softmax(logits, axis=-1)
    gate_score, gate_idx = jax.lax.top_k(probs, _TOPK)
    wcomb = gate_score * alpha[gate_idx]

    eid = gate_idx.reshape(-1).astype(jnp.int32)
    wgt = wcomb.reshape(-1)
    tok = jnp.arange(_NP, dtype=jnp.int32) // _TOPK
    onehot = (eid[:, None] == jnp.arange(_E, dtype=jnp.int32)[None, :])
    onehot = onehot.astype(jnp.int32)
    counts = onehot.sum(axis=0)
    nblk_e = (counts + _BLK - 1) // _BLK
    bounds = jnp.cumsum(nblk_e)
    pad_off = jnp.concatenate(
        [jnp.zeros((1,), jnp.int32), bounds[:-1].astype(jnp.int32)]) * _BLK
    rank = jnp.cumsum(onehot, axis=0) - onehot
    rank_pair = jnp.sum(rank * onehot, axis=1)
    pos = pad_off[eid] + rank_pair
    row_token = jnp.zeros((_P,), jnp.int32).at[pos].set(tok)
    row_weight = jnp.zeros((_P,), jnp.float32).at[pos].set(wgt)
    n_act = bounds[-1].astype(jnp.int32)
    jblk = jnp.arange(_NB, dtype=jnp.int32)
    be_full = (bounds[None, :] <= jblk[:, None]).sum(axis=1).astype(jnp.int32)
    be_last = (bounds <= (n_act - 1)).sum().astype(jnp.int32)
    be = jnp.where(jblk < n_act, jnp.minimum(be_full, _E - 1), be_last)
    meta = jnp.concatenate([be, n_act[None]]).astype(jnp.int32)

    gather = _make_sc_gather()
    xs_a = gather(x, row_token[:_PH])
    xs_b = gather(x, row_token[_PH:])

    wrow = jnp.broadcast_to(row_weight[:, None], (_P, 128))
    w2b = _cast_w2(fc2_w)
    args = (fc1_w, fc1_b.reshape(_E, 1, _DFF),
            w2b, fc2_b.reshape(_E, 1, _D),
            ln_scale.reshape(_E, 1, _D), ln_bias.reshape(_E, 1, _D))
    y_a = _ffn_half(0, meta, xs_a, wrow, *args)
    y_b = _ffn_half(1, meta, xs_b, wrow, *args)
    y = jnp.concatenate([y_a, y_b], axis=0)

    pos2 = pos.reshape(_S, _TOPK)
    out = _make_sc_combine()(y, pos2[:, 0], pos2[:, 1])
    return out.reshape(_B, _S, _D)


# speedup vs baseline: 1.0150x; 1.0150x over previous
import functools

import jax
import jax.numpy as jnp
from jax import lax
from jax.experimental import pallas as pl
from jax.experimental.pallas import tpu as pltpu
from jax.experimental.pallas import tpu_sc as plsc

_B, _S, _D, _DFF, _E, _TOPK = 1, 2048, 1024, 4096, 8, 2
_EPS = 1e-5
_BLK = 128
_NP = _S * _TOPK
_P = _NP + _E * _BLK
_NB = _P // _BLK
_NC, _NS = 2, 16
_NW = _NC * _NS


def _wid():
    return lax.axis_index("s") * _NC + lax.axis_index("c")


_PH = _P // 2
_GPW = _PH // _NW
_GCH = _GPW // 2


@functools.cache
def _make_sc_gather():
    @functools.partial(
        pl.kernel,
        mesh=plsc.VectorSubcoreMesh(core_axis_name="c", subcore_axis_name="s"),
        out_type=jax.ShapeDtypeStruct((_PH, _D), jnp.float32),
        scratch_types=[
            pltpu.VMEM((_GPW,), jnp.int32),
            pltpu.VMEM((_GCH, _D), jnp.float32),
            pltpu.VMEM((_GCH, _D), jnp.float32),
            pltpu.SemaphoreType.DMA,
            pltpu.SemaphoreType.DMA,
        ],
    )
    def _sc_gather(x_hbm, rt_hbm, xs_hbm, idx_v, b0, b1, s0, s1):
        w = _wid()
        base = w * _GPW
        pltpu.sync_copy(rt_hbm.at[pl.ds(base, _GPW)], idx_v)
        bufs, sems = (b0, b1), (s0, s1)
        cps = [None, None]
        for c in range(2):
            cps[c % 2] = pltpu.async_copy(
                x_hbm.at[idx_v.at[pl.ds(c * _GCH, _GCH)]],
                bufs[c % 2], sems[c % 2])
            if c >= 1:
                cps[(c - 1) % 2].wait()
                pltpu.sync_copy(bufs[(c - 1) % 2],
                                xs_hbm.at[pl.ds(base + (c - 1) * _GCH, _GCH)])
        cps[1].wait()
        pltpu.sync_copy(bufs[1], xs_hbm.at[pl.ds(base + _GCH, _GCH)])

    return _sc_gather


_CCH = 32


@functools.cache
def _make_sc_combine():
    @functools.partial(
        pl.kernel,
        mesh=plsc.VectorSubcoreMesh(core_axis_name="c", subcore_axis_name="s"),
        out_type=jax.ShapeDtypeStruct((_S, _D), jnp.float32),
        scratch_types=[
            pltpu.VMEM((_CCH,), jnp.int32),
            pltpu.VMEM((_CCH,), jnp.int32),
            pltpu.VMEM((_CCH, _D), jnp.float32),
            pltpu.VMEM((_CCH, _D), jnp.float32),
            pltpu.SemaphoreType.DMA,
            pltpu.SemaphoreType.DMA,
        ],
    )
    def _sc_combine(y_hbm, p0_hbm, p1_hbm, out_hbm, i0_v, i1_v, a_v, b_v,
                    s0, s1):
        w = _wid()
        for c in range(2):
            base = w * (2 * _CCH) + c * _CCH
            pltpu.sync_copy(p0_hbm.at[pl.ds(base, _CCH)], i0_v)
            pltpu.sync_copy(p1_hbm.at[pl.ds(base, _CCH)], i1_v)
            cp0 = pltpu.async_copy(y_hbm.at[i0_v], a_v, s0)
            cp1 = pltpu.async_copy(y_hbm.at[i1_v], b_v, s1)
            cp0.wait()
            cp1.wait()

            def _row(r, carry):
                for j in range(_D // 16):
                    sl = pl.ds(j * 16, 16)
                    a_v[r, sl] = a_v[r, sl] + b_v[r, sl]
                return carry

            lax.fori_loop(0, _CCH, _row, 0)
            pltpu.sync_copy(a_v, out_hbm.at[pl.ds(base, _CCH)])

    return _sc_combine


def _cast_body(b_ref, bo_ref):
    bo_ref[...] = b_ref[...].astype(jnp.bfloat16)


def _cast_w2(w2):
    w2f = w2.reshape(_E * _DFF, _D)
    n = 8
    r2 = w2f.shape[0] // n
    o2 = pl.pallas_call(
        _cast_body,
        grid=(n,),
        in_specs=[pl.BlockSpec((r2, _D), lambda i: (i, 0))],
        out_specs=pl.BlockSpec((r2, _D), lambda i: (i, 0)),
        out_shape=jax.ShapeDtypeStruct(w2f.shape, jnp.bfloat16),
        compiler_params=pltpu.CompilerParams(
            dimension_semantics=("arbitrary",)),
    )(w2f)
    return o2.reshape(_E, _DFF, _D)


_NBH = _NB // 2


def _ffn_body(off, meta_ref, xs_ref, wrow_ref, w1_ref, b1_ref, w2_ref,
              b2_ref, ls_ref, lb_ref, y_ref):
    i = pl.program_id(0) + off
    n_act = meta_ref[_NB]

    @pl.when(i < n_act)
    def _():
        x = xs_ref[...].astype(jnp.float32)
        mu = jnp.mean(x, axis=1, keepdims=True)
        xc = x - mu
        var = jnp.mean(xc * xc, axis=1, keepdims=True)
        xn = xc * lax.rsqrt(var + _EPS)
        h = xn * ls_ref[0] + lb_ref[0]
        a = jnp.dot(h, w1_ref[0],
                    preferred_element_type=jnp.float32) + b1_ref[0]
        g = jax.nn.gelu(a)
        y = jnp.dot(g.astype(jnp.bfloat16), w2_ref[0],
                    preferred_element_type=jnp.float32) + b2_ref[0]
        y_ref[...] = y * wrow_ref[:, :1]


def _ffn_half(off, meta, xs, wrow, w1, b1, w2, b2, ls, lb):
    grid_spec = pltpu.PrefetchScalarGridSpec(
        num_scalar_prefetch=1,
        grid=(_NBH,),
        in_specs=[
            pl.BlockSpec((_BLK, _D), lambda i, m: (i, 0)),
            pl.BlockSpec((_BLK, 128), lambda i, m: (i + off * _NBH, 0)),
            pl.BlockSpec((1, _D, _DFF), lambda i, m: (m[i + off * _NBH], 0, 0)),
            pl.BlockSpec((1, 1, _DFF), lambda i, m: (m[i + off * _NBH], 0, 0)),
            pl.BlockSpec((1, _DFF, _D), lambda i, m: (m[i + off * _NBH], 0, 0)),
            pl.BlockSpec((1, 1, _D), lambda i, m: (m[i + off * _NBH], 0, 0)),
            pl.BlockSpec((1, 1, _D), lambda i, m: (m[i + off * _NBH], 0, 0)),
            pl.BlockSpec((1, 1, _D), lambda i, m: (m[i + off * _NBH], 0, 0)),
        ],
        out_specs=pl.BlockSpec((_BLK, _D), lambda i, m: (i, 0)),
    )
    return pl.pallas_call(
        functools.partial(_ffn_body, off * _NBH),
        grid_spec=grid_spec,
        out_shape=jax.ShapeDtypeStruct((_PH, _D), jnp.float32),
        compiler_params=pltpu.CompilerParams(
            dimension_semantics=("arbitrary",)),
    )(meta, xs, wrow, w1, b1, w2, b2, ls, lb)


def kernel(hidden_states, gate_w, alpha, ln_scale, ln_bias, fc1_w, fc1_b,
           fc2_w, fc2_b):
    x = hidden_states.reshape(_S, _D)

    logits = x @ gate_w
    probs = jax.nn.softmax(logits, axis=-1)
    gate_score, gate_idx = jax.lax.top_k(probs, _TOPK)
    wcomb = gate_score * alpha[gate_idx]

    eid = gate_idx.reshape(-1).astype(jnp.int32)
    wgt = wcomb.reshape(-1)
    tok = jnp.arange(_NP, dtype=jnp.int32) // _TOPK
    onehot = (eid[:, None] == jnp.arange(_E, dtype=jnp.int32)[None, :])
    onehot = onehot.astype(jnp.int32)
    counts = onehot.sum(axis=0)
    nblk_e = (counts + _BLK - 1) // _BLK
    bounds = jnp.cumsum(nblk_e)
    pad_off = jnp.concatenate(
        [jnp.zeros((1,), jnp.int32), bounds[:-1].astype(jnp.int32)]) * _BLK
    rank = jnp.cumsum(onehot, axis=0) - onehot
    rank_pair = jnp.sum(rank * onehot, axis=1)
    pos = pad_off[eid] + rank_pair
    row_token = jnp.zeros((_P,), jnp.int32).at[pos].set(tok)
    row_weight = jnp.zeros((_P,), jnp.float32).at[pos].set(wgt)
    n_act = bounds[-1].astype(jnp.int32)
    jblk = jnp.arange(_NB, dtype=jnp.int32)
    be_full = (bounds[None, :] <= jblk[:, None]).sum(axis=1).astype(jnp.int32)
    be_last = (bounds <= (n_act - 1)).sum().astype(jnp.int32)
    be = jnp.where(jblk < n_act, jnp.minimum(be_full, _E - 1), be_last)
    meta = jnp.concatenate([be, n_act[None]]).astype(jnp.int32)

    gather = _make_sc_gather()
    xs_a = gather(x, row_token[:_PH])
    xs_b = gather(x, row_token[_PH:])

    wrow = jnp.broadcast_to(row_weight[:, None], (_P, 128))
    w2b = _cast_w2(fc2_w)
    args = (fc1_w, fc1_b.reshape(_E, 1, _DFF),
            w2b, fc2_b.reshape(_E, 1, _D),
            ln_scale.reshape(_E, 1, _D), ln_bias.reshape(_E, 1, _D))
    y_a = _ffn_half(0, meta, xs_a, wrow, *args)
    y_b = _ffn_half(1, meta, xs_b, wrow, *args)
    y = jnp.concatenate([y_a, y_b], axis=0)

    pos2 = pos.reshape(_S, _TOPK)
    out = _make_sc_combine()(y, pos2[:, 0], pos2[:, 1])
    return out.reshape(_B, _S, _D)


# speedup vs baseline: 1.0165x; 1.0015x over previous
import functools

import jax
import jax.numpy as jnp
from jax import lax
from jax.experimental import pallas as pl
from jax.experimental.pallas import tpu as pltpu
from jax.experimental.pallas import tpu_sc as plsc

_B, _S, _D, _DFF, _E, _TOPK = 1, 2048, 1024, 4096, 8, 2
_EPS = 1e-5
_BLK = 128
_NP = _S * _TOPK
_P = _NP + _E * _BLK
_NB = _P // _BLK
_NC, _NS = 2, 16
_NW = _NC * _NS


def _wid():
    return lax.axis_index("s") * _NC + lax.axis_index("c")


_PH = _P // 2
_GPW = _PH // _NW
_GCH = _GPW // 2


@functools.cache
def _make_sc_gather():
    @functools.partial(
        pl.kernel,
        mesh=plsc.VectorSubcoreMesh(core_axis_name="c", subcore_axis_name="s"),
        out_type=jax.ShapeDtypeStruct((_PH, _D), jnp.float32),
        scratch_types=[
            pltpu.VMEM((_GPW,), jnp.int32),
            pltpu.VMEM((_GCH, _D), jnp.float32),
            pltpu.VMEM((_GCH, _D), jnp.float32),
            pltpu.SemaphoreType.DMA,
            pltpu.SemaphoreType.DMA,
        ],
    )
    def _sc_gather(x_hbm, rt_hbm, xs_hbm, idx_v, b0, b1, s0, s1):
        w = _wid()
        base = w * _GPW
        pltpu.sync_copy(rt_hbm.at[pl.ds(base, _GPW)], idx_v)
        bufs, sems = (b0, b1), (s0, s1)
        cps = [None, None]
        for c in range(2):
            cps[c % 2] = pltpu.async_copy(
                x_hbm.at[idx_v.at[pl.ds(c * _GCH, _GCH)]],
                bufs[c % 2], sems[c % 2])
            if c >= 1:
                cps[(c - 1) % 2].wait()
                pltpu.sync_copy(bufs[(c - 1) % 2],
                                xs_hbm.at[pl.ds(base + (c - 1) * _GCH, _GCH)])
        cps[1].wait()
        pltpu.sync_copy(bufs[1], xs_hbm.at[pl.ds(base + _GCH, _GCH)])

    return _sc_gather


_CCH = 32


@functools.cache
def _make_sc_combine():
    @functools.partial(
        pl.kernel,
        mesh=plsc.VectorSubcoreMesh(core_axis_name="c", subcore_axis_name="s"),
        out_type=jax.ShapeDtypeStruct((_S, _D), jnp.float32),
        scratch_types=[
            pltpu.VMEM((_CCH,), jnp.int32),
            pltpu.VMEM((_CCH,), jnp.int32),
            pltpu.VMEM((_CCH, _D), jnp.float32),
            pltpu.VMEM((_CCH, _D), jnp.float32),
            pltpu.VMEM((_CCH, 16), jnp.float32),
            pltpu.VMEM((_CCH, 16), jnp.float32),
            pltpu.SemaphoreType.DMA,
            pltpu.SemaphoreType.DMA,
        ],
    )
    def _sc_combine(y_hbm, p0_hbm, p1_hbm, cw0_hbm, cw1_hbm, out_hbm,
                    i0_v, i1_v, a_v, b_v, w0_v, w1_v, s0, s1):
        w = _wid()
        for c in range(2):
            base = w * (2 * _CCH) + c * _CCH
            pltpu.sync_copy(p0_hbm.at[pl.ds(base, _CCH)], i0_v)
            pltpu.sync_copy(p1_hbm.at[pl.ds(base, _CCH)], i1_v)
            pltpu.sync_copy(cw0_hbm.at[pl.ds(base, _CCH)], w0_v)
            pltpu.sync_copy(cw1_hbm.at[pl.ds(base, _CCH)], w1_v)
            cp0 = pltpu.async_copy(y_hbm.at[i0_v], a_v, s0)
            cp1 = pltpu.async_copy(y_hbm.at[i1_v], b_v, s1)
            cp0.wait()
            cp1.wait()

            def _row(r, carry):
                s0v = w0_v[r, :]
                s1v = w1_v[r, :]
                for j in range(_D // 16):
                    sl = pl.ds(j * 16, 16)
                    a_v[r, sl] = a_v[r, sl] * s0v + b_v[r, sl] * s1v
                return carry

            lax.fori_loop(0, _CCH, _row, 0)
            pltpu.sync_copy(a_v, out_hbm.at[pl.ds(base, _CCH)])

    return _sc_combine


def _cast_body(b_ref, bo_ref):
    bo_ref[...] = b_ref[...].astype(jnp.bfloat16)


def _cast_w2(w2):
    w2f = w2.reshape(_E * _DFF, _D)
    n = 8
    r2 = w2f.shape[0] // n
    o2 = pl.pallas_call(
        _cast_body,
        grid=(n,),
        in_specs=[pl.BlockSpec((r2, _D), lambda i: (i, 0))],
        out_specs=pl.BlockSpec((r2, _D), lambda i: (i, 0)),
        out_shape=jax.ShapeDtypeStruct(w2f.shape, jnp.bfloat16),
        compiler_params=pltpu.CompilerParams(
            dimension_semantics=("arbitrary",)),
    )(w2f)
    return o2.reshape(_E, _DFF, _D)


_NBH = _NB // 2


def _ffn_body(off, meta_ref, xs_ref, w1_ref, b1_ref, w2_ref,
              b2_ref, ls_ref, lb_ref, y_ref):
    i = pl.program_id(0) + off
    n_act = meta_ref[_NB]

    @pl.when(i < n_act)
    def _():
        x = xs_ref[...].astype(jnp.float32)
        mu = jnp.mean(x, axis=1, keepdims=True)
        xc = x - mu
        var = jnp.mean(xc * xc, axis=1, keepdims=True)
        xn = xc * lax.rsqrt(var + _EPS)
        h = xn * ls_ref[0] + lb_ref[0]
        a = jnp.dot(h, w1_ref[0],
                    preferred_element_type=jnp.float32) + b1_ref[0]
        g = jax.nn.gelu(a)
        y = jnp.dot(g.astype(jnp.bfloat16), w2_ref[0],
                    preferred_element_type=jnp.float32) + b2_ref[0]
        y_ref[...] = y


def _ffn_half(off, meta, xs, w1, b1, w2, b2, ls, lb):
    grid_spec = pltpu.PrefetchScalarGridSpec(
        num_scalar_prefetch=1,
        grid=(_NBH,),
        in_specs=[
            pl.BlockSpec((_BLK, _D), lambda i, m: (i, 0)),
            pl.BlockSpec((1, _D, _DFF), lambda i, m: (m[i + off * _NBH], 0, 0)),
            pl.BlockSpec((1, 1, _DFF), lambda i, m: (m[i + off * _NBH], 0, 0)),
            pl.BlockSpec((1, _DFF, _D), lambda i, m: (m[i + off * _NBH], 0, 0)),
            pl.BlockSpec((1, 1, _D), lambda i, m: (m[i + off * _NBH], 0, 0)),
            pl.BlockSpec((1, 1, _D), lambda i, m: (m[i + off * _NBH], 0, 0)),
            pl.BlockSpec((1, 1, _D), lambda i, m: (m[i + off * _NBH], 0, 0)),
        ],
        out_specs=pl.BlockSpec((_BLK, _D), lambda i, m: (i, 0)),
    )
    return pl.pallas_call(
        functools.partial(_ffn_body, off * _NBH),
        grid_spec=grid_spec,
        out_shape=jax.ShapeDtypeStruct((_PH, _D), jnp.float32),
        compiler_params=pltpu.CompilerParams(
            dimension_semantics=("arbitrary",)),
    )(meta, xs, w1, b1, w2, b2, ls, lb)


def kernel(hidden_states, gate_w, alpha, ln_scale, ln_bias, fc1_w, fc1_b,
           fc2_w, fc2_b):
    x = hidden_states.reshape(_S, _D)

    logits = x @ gate_w
    probs = jax.nn.softmax(logits, axis=-1)
    gate_score, gate_idx = jax.lax.top_k(probs, _TOPK)
    wcomb = gate_score * alpha[gate_idx]

    eid = gate_idx.reshape(-1).astype(jnp.int32)
    wgt = wcomb.reshape(-1)
    tok = jnp.arange(_NP, dtype=jnp.int32) // _TOPK
    onehot = (eid[:, None] == jnp.arange(_E, dtype=jnp.int32)[None, :])
    onehot = onehot.astype(jnp.int32)
    counts = onehot.sum(axis=0)
    nblk_e = (counts + _BLK - 1) // _BLK
    bounds = jnp.cumsum(nblk_e)
    pad_off = jnp.concatenate(
        [jnp.zeros((1,), jnp.int32), bounds[:-1].astype(jnp.int32)]) * _BLK
    rank = jnp.cumsum(onehot, axis=0) - onehot
    rank_pair = jnp.sum(rank * onehot, axis=1)
    pos = pad_off[eid] + rank_pair
    row_token = jnp.zeros((_P,), jnp.int32).at[pos].set(tok)
    n_act = bounds[-1].astype(jnp.int32)
    jblk = jnp.arange(_NB, dtype=jnp.int32)
    be_full = (bounds[None, :] <= jblk[:, None]).sum(axis=1).astype(jnp.int32)
    be_last = (bounds <= (n_act - 1)).sum().astype(jnp.int32)
    be = jnp.where(jblk < n_act, jnp.minimum(be_full, _E - 1), be_last)
    meta = jnp.concatenate([be, n_act[None]]).astype(jnp.int32)

    gather = _make_sc_gather()
    xs_a = gather(x, row_token[:_PH])
    xs_b = gather(x, row_token[_PH:])

    w2b = _cast_w2(fc2_w)
    args = (fc1_w, fc1_b.reshape(_E, 1, _DFF),
            w2b, fc2_b.reshape(_E, 1, _D),
            ln_scale.reshape(_E, 1, _D), ln_bias.reshape(_E, 1, _D))
    y_a = _ffn_half(0, meta, xs_a, *args)
    y_b = _ffn_half(1, meta, xs_b, *args)
    y = jnp.concatenate([y_a, y_b], axis=0)

    pos2 = pos.reshape(_S, _TOPK)
    cw0 = jnp.broadcast_to(wcomb[:, 0:1], (_S, 16))
    cw1 = jnp.broadcast_to(wcomb[:, 1:2], (_S, 16))
    out = _make_sc_combine()(y, pos2[:, 0], pos2[:, 1], cw0, cw1)
    return out.reshape(_B, _S, _D)
